# baseline (speedup 1.0000x reference)
import functools

import jax
import jax.numpy as jnp
from jax import lax
from jax.experimental import pallas as pl
from jax.experimental.pallas import tpu as pltpu
from jax.experimental.pallas import tpu_sc as plsc

UNITS_X = 512
UNITS_Y = 512
N_CLASSES = 64
BATCH = 16384


_ROWS_PER_BLOCK = 16


def _table_body(cc_ref, out_ref):
    cc = cc_ref[...]
    s = jnp.sum(cc, axis=-1)
    m = jnp.max(cc, axis=-1, keepdims=True)
    iota = lax.broadcasted_iota(jnp.int32, cc.shape, 2)
    idx = jnp.min(jnp.where(cc == m, iota, N_CLASSES), axis=-1)
    out_ref[...] = jnp.where(s == 0.0, -1.0, idx.astype(jnp.float32))


def _predicted_class_table(class_count):
    grid = (UNITS_X // _ROWS_PER_BLOCK,)
    return pl.pallas_call(
        _table_body,
        grid=grid,
        in_specs=[pl.BlockSpec(
            (_ROWS_PER_BLOCK, UNITS_Y, N_CLASSES), lambda i: (i, 0, 0))],
        out_specs=pl.BlockSpec((_ROWS_PER_BLOCK, UNITS_Y), lambda i: (i, 0)),
        out_shape=jax.ShapeDtypeStruct((UNITS_X, UNITS_Y), jnp.float32),
    )(class_count)


_NC = 2
_NS = 16
_NW = _NC * _NS
_BPW = BATCH // _NW
_CH = 64

@functools.lru_cache(maxsize=None)
def _build_sc_gather():
    mesh = plsc.VectorSubcoreMesh(core_axis_name="c", subcore_axis_name="s")

    @functools.partial(
        pl.kernel,
        mesh=mesh,
        out_type=jax.ShapeDtypeStruct((BATCH, UNITS_Y), jnp.float32),
        scratch_types=[
            pltpu.VMEM((_BPW,), jnp.int32),
            pltpu.VMEM((_BPW,), jnp.int32),
            pltpu.VMEM((_CH, UNITS_Y), jnp.float32),
            pltpu.SemaphoreType.DMA,
            pltpu.SemaphoreType.DMA,
        ],
    )
    def _sc_gather(table_hbm, x0_hbm, x1_hbm, out_hbm, x1_v, idx_v, rows_v,
                   sem_idx, sem_rows):
        wid = lax.axis_index("s") * _NC + lax.axis_index("c")
        base = wid * _BPW
        pltpu.sync_copy(x1_hbm.at[pl.ds(base, _BPW)], x1_v)
        pltpu.async_copy(x0_hbm.at[x1_v], idx_v, sem_idx).wait()

        def body(i, carry):
            off = i * _CH
            pltpu.async_copy(
                table_hbm.at[idx_v.at[pl.ds(off, _CH)]], rows_v,
                sem_rows).wait()
            pltpu.sync_copy(rows_v, out_hbm.at[pl.ds(base + off, _CH)])
            return carry

        lax.fori_loop(0, _BPW // _CH, body, 0)

    return _sc_gather


def kernel(class_count, x):
    table = _predicted_class_table(class_count)
    x = x.astype(jnp.int32)
    return _build_sc_gather()(table, x[0], x[1])


# speedup vs baseline: 1.4207x; 1.4207x over previous
import functools

import jax
import jax.numpy as jnp
from jax import lax
from jax.experimental import pallas as pl
from jax.experimental.pallas import tpu as pltpu
from jax.experimental.pallas import tpu_sc as plsc

UNITS_X = 512
UNITS_Y = 512
N_CLASSES = 64
BATCH = 16384


_ROWS_PER_BLOCK = 16


def _reduce_body(cc_ref, out_ref):
    cc = cc_ref[...]
    rev_i = lax.broadcasted_iota(jnp.int32, (1, 1, N_CLASSES), 2)
    rev = (float(N_CLASSES - 1) - rev_i.astype(jnp.float32)) * (
        1.0 / N_CLASSES)
    key = cc + rev
    out_ref[...] = jnp.max(key, axis=-1)


def _decode_body(m_ref, out_ref):
    ki = (m_ref[...] * float(N_CLASSES)).astype(jnp.int32)
    label = (float(N_CLASSES - 1) - (ki & (N_CLASSES - 1)).astype(jnp.float32))
    out_ref[...] = jnp.where(ki < N_CLASSES, -1.0, label)


def _predicted_class_table(class_count):
    grid = (UNITS_X // _ROWS_PER_BLOCK,)
    m = pl.pallas_call(
        _reduce_body,
        grid=grid,
        in_specs=[pl.BlockSpec(
            (_ROWS_PER_BLOCK, UNITS_Y, N_CLASSES), lambda i: (i, 0, 0))],
        out_specs=pl.BlockSpec((_ROWS_PER_BLOCK, UNITS_Y), lambda i: (i, 0)),
        out_shape=jax.ShapeDtypeStruct((UNITS_X, UNITS_Y), jnp.float32),
    )(class_count)
    return pl.pallas_call(
        _decode_body,
        out_shape=jax.ShapeDtypeStruct((UNITS_X, UNITS_Y), jnp.float32),
    )(m)


_NC = 2
_NS = 16
_NW = _NC * _NS
_BPW = BATCH // _NW
_CH = 64

@functools.lru_cache(maxsize=None)
def _build_sc_gather():
    mesh = plsc.VectorSubcoreMesh(core_axis_name="c", subcore_axis_name="s")

    @functools.partial(
        pl.kernel,
        mesh=mesh,
        out_type=jax.ShapeDtypeStruct((BATCH, UNITS_Y), jnp.float32),
        scratch_types=[
            pltpu.VMEM((_BPW,), jnp.int32),
            pltpu.VMEM((_BPW,), jnp.int32),
            pltpu.VMEM((_CH, UNITS_Y), jnp.float32),
            pltpu.SemaphoreType.DMA,
            pltpu.SemaphoreType.DMA,
        ],
    )
    def _sc_gather(table_hbm, x0_hbm, x1_hbm, out_hbm, x1_v, idx_v, rows_v,
                   sem_idx, sem_rows):
        wid = lax.axis_index("s") * _NC + lax.axis_index("c")
        base = wid * _BPW
        pltpu.sync_copy(x1_hbm.at[pl.ds(base, _BPW)], x1_v)
        pltpu.async_copy(x0_hbm.at[x1_v], idx_v, sem_idx).wait()

        def body(i, carry):
            off = i * _CH
            pltpu.async_copy(
                table_hbm.at[idx_v.at[pl.ds(off, _CH)]], rows_v,
                sem_rows).wait()
            pltpu.sync_copy(rows_v, out_hbm.at[pl.ds(base + off, _CH)])
            return carry

        lax.fori_loop(0, _BPW // _CH, body, 0)

    return _sc_gather


def kernel(class_count, x):
    table = _predicted_class_table(class_count)
    x = x.astype(jnp.int32)
    return _build_sc_gather()(table, x[0], x[1])


# speedup vs baseline: 2.6878x; 1.8918x over previous
import functools

import jax
import jax.numpy as jnp
from jax import lax
from jax.experimental import pallas as pl
from jax.experimental.pallas import tpu as pltpu
from jax.experimental.pallas import tpu_sc as plsc

UNITS_X = 512
UNITS_Y = 512
N_CLASSES = 64
BATCH = 16384


_ROWS_PER_BLOCK = 16


def _reduce_body(cc_ref, out_ref):
    cc = cc_ref[...]
    rev_i = lax.broadcasted_iota(jnp.int32, (1, N_CLASSES, 1), 1)
    rev = (float(N_CLASSES - 1) - rev_i.astype(jnp.float32)) * (
        1.0 / N_CLASSES)
    key = cc + rev
    out_ref[...] = jnp.max(key, axis=1)


def _decode_body(m_ref, out_ref):
    ki = (m_ref[...] * float(N_CLASSES)).astype(jnp.int32)
    label = (float(N_CLASSES - 1) - (ki & (N_CLASSES - 1)).astype(jnp.float32))
    out_ref[...] = jnp.where(ki < N_CLASSES, -1.0, label)


def _predicted_class_table(class_count):
    cc_t = jnp.transpose(class_count, (0, 2, 1))
    grid = (UNITS_X // _ROWS_PER_BLOCK,)
    m = pl.pallas_call(
        _reduce_body,
        grid=grid,
        in_specs=[pl.BlockSpec(
            (_ROWS_PER_BLOCK, N_CLASSES, UNITS_Y), lambda i: (i, 0, 0))],
        out_specs=pl.BlockSpec((_ROWS_PER_BLOCK, UNITS_Y), lambda i: (i, 0)),
        out_shape=jax.ShapeDtypeStruct((UNITS_X, UNITS_Y), jnp.float32),
    )(cc_t)
    return pl.pallas_call(
        _decode_body,
        out_shape=jax.ShapeDtypeStruct((UNITS_X, UNITS_Y), jnp.float32),
    )(m)


_NC = 2
_NS = 16
_NW = _NC * _NS
_BPW = BATCH // _NW
_CH = 64

@functools.lru_cache(maxsize=None)
def _build_sc_gather():
    mesh = plsc.VectorSubcoreMesh(core_axis_name="c", subcore_axis_name="s")

    @functools.partial(
        pl.kernel,
        mesh=mesh,
        out_type=jax.ShapeDtypeStruct((BATCH, UNITS_Y), jnp.float32),
        scratch_types=[
            pltpu.VMEM((_BPW,), jnp.int32),
            pltpu.VMEM((_BPW,), jnp.int32),
            pltpu.VMEM((_CH, UNITS_Y), jnp.float32),
            pltpu.SemaphoreType.DMA,
            pltpu.SemaphoreType.DMA,
        ],
    )
    def _sc_gather(table_hbm, x0_hbm, x1_hbm, out_hbm, x1_v, idx_v, rows_v,
                   sem_idx, sem_rows):
        wid = lax.axis_index("s") * _NC + lax.axis_index("c")
        base = wid * _BPW
        pltpu.sync_copy(x1_hbm.at[pl.ds(base, _BPW)], x1_v)
        pltpu.async_copy(x0_hbm.at[x1_v], idx_v, sem_idx).wait()

        def body(i, carry):
            off = i * _CH
            pltpu.async_copy(
                table_hbm.at[idx_v.at[pl.ds(off, _CH)]], rows_v,
                sem_rows).wait()
            pltpu.sync_copy(rows_v, out_hbm.at[pl.ds(base + off, _CH)])
            return carry

        lax.fori_loop(0, _BPW // _CH, body, 0)

    return _sc_gather


def kernel(class_count, x):
    table = _predicted_class_table(class_count)
    x = x.astype(jnp.int32)
    return _build_sc_gather()(table, x[0], x[1])
